# Initial kernel scaffold; baseline (speedup 1.0000x reference)
import jax
import jax.numpy as jnp
from jax.experimental import pallas as pl


def kernel(patch_features):
    raise NotImplementedError("write your pallas kernel here")


# speedup vs baseline: 1.3417x; 1.3417x over previous
import math

import jax
import jax.numpy as jnp
from jax.experimental import pallas as pl
from jax.experimental.pallas import tpu as pltpu

B, N, C = 32, 576, 768
CLUSTER_NUMS = (144, 72, 36)
KTOT = sum(CLUSTER_NUMS)
T = 96
NT = N // T


def _stage1_body(dist_in_ref, dens_ref, scores_ref):
    dmax = jnp.float32(0.0)
    for t in range(NT):
        dist = dist_in_ref[0, t * T:(t + 1) * T, :]
        dmax = jnp.maximum(dmax, jnp.max(dist))

    for r in range(3):
        dens_row = dens_ref[0, r][None, :]
        for t in range(NT):
            dist = dist_in_ref[0, t * T:(t + 1) * T, :]
            dr = dens_row[0, t * T:(t + 1) * T]
            tmp = jnp.where(dens_row > dr[:, None], dist, dmax)
            peak = jnp.min(tmp, axis=1)
            scores_ref[0, r, t * T:(t + 1) * T] = peak * dr


def _stage1(dist, dens):
    return pl.pallas_call(
        _stage1_body,
        grid=(B,),
        in_specs=[
            pl.BlockSpec((1, N, N), lambda b: (b, 0, 0)),
            pl.BlockSpec((1, 3, N), lambda b: (b, 0, 0)),
        ],
        out_specs=pl.BlockSpec((1, 3, N), lambda b: (b, 0, 0)),
        out_shape=jax.ShapeDtypeStruct((B, 3, N), jnp.float32),
    )(dist, dens)


def _stage2_body(scores_ref, centers_ref):
    iota = jax.lax.broadcasted_iota(jnp.int32, (B, N), 1)
    neg = jnp.float32(-3.4e38)

    def topk(score, k):
        col = jax.lax.broadcasted_iota(jnp.int32, (B, k), 1)

        def step(t, carry):
            cur, idxs = carry
            m = jnp.max(cur, axis=1)
            hit = cur == m[:, None]
            am = jnp.min(jnp.where(hit, iota, N), axis=1)
            cur = jnp.where(iota == am[:, None], neg, cur)
            idxs = jnp.where(col == t, am[:, None], idxs)
            return cur, idxs

        _, idxs = jax.lax.fori_loop(
            0, k, step, (score, jnp.zeros((B, k), jnp.int32)))
        return idxs

    parts = [topk(scores_ref[:, r, :], CLUSTER_NUMS[r]) for r in range(3)]
    centers_ref[...] = jnp.concatenate(parts, axis=1)


def _stage2(scores):
    return pl.pallas_call(
        _stage2_body,
        out_shape=jax.ShapeDtypeStruct((B, KTOT), jnp.int32),
    )(scores)


def _stage3_body(x_ref, dist_ref, centers_ref, merged_ref):
    x = x_ref[0]
    dist = dist_ref[0]
    iota_n = jax.lax.broadcasted_iota(jnp.int32, (1, N), 1)
    off = 0
    for r, k in enumerate(CLUSTER_NUMS):
        idxs = centers_ref[0, 0, off:off + k]
        iota_k = jax.lax.broadcasted_iota(jnp.int32, (k, 1), 0)
        onehot_b = idxs[:, None] == iota_n
        onehot = onehot_b.astype(jnp.float32)
        dist_down = jnp.dot(onehot, dist,
                            preferred_element_type=jnp.float32,
                            precision=jax.lax.Precision.HIGHEST)
        m = jnp.min(dist_down, axis=0)
        amin = jnp.min(jnp.where(dist_down == m[None, :], iota_k, k),
                       axis=0)
        rank = jnp.max(jnp.where(onehot_b, iota_k, -1), axis=0)
        assign = jnp.where(rank >= 0, rank, amin)
        onehot_a = (assign[None, :] == iota_k).astype(jnp.float32)
        cnt = jnp.sum(onehot_a, axis=1)
        sums = jnp.dot(onehot_a, x, preferred_element_type=jnp.float32,
                       precision=jax.lax.Precision.HIGHEST)
        merged_ref[0, off:off + k] = sums * (1.0 / (cnt + 1e-6))[:, None]
        off += k


def _stage3(x, dist, centers3):
    return pl.pallas_call(
        _stage3_body,
        grid=(B,),
        in_specs=[
            pl.BlockSpec((1, N, C), lambda b: (b, 0, 0)),
            pl.BlockSpec((1, N, N), lambda b: (b, 0, 0)),
            pl.BlockSpec((1, 1, KTOT), lambda b: (b, 0, 0)),
        ],
        out_specs=pl.BlockSpec((1, KTOT, C), lambda b: (b, 0, 0)),
        out_shape=jax.ShapeDtypeStruct((B, KTOT, C), jnp.float32),
    )(x, dist, centers3)


def kernel(patch_features):
    x = patch_features
    nkey = jax.random.key(42)
    noise = jnp.stack([
        jax.random.uniform(jax.random.fold_in(nkey, i), (B, N),
                           dtype=patch_features.dtype)
        for i in range(3)], axis=1)
    x2 = jnp.sum(x * x, axis=-1)
    d2 = x2[:, :, None] + x2[:, None, :] - 2.0 * jnp.einsum('bnc,bmc->bnm', x, x)
    dist = jnp.sqrt(jnp.maximum(d2, 0.0)) / (C ** 0.5)
    dens_list = []
    for r, k in enumerate((5, 3, 3)):
        neg_vals, _ = jax.lax.top_k(-dist, k)
        dens_list.append(jnp.exp(-jnp.mean((-neg_vals) ** 2, axis=-1))
                         + noise[:, r, :] * 1e-6)
    dens = jnp.stack(dens_list, axis=1)
    scores = _stage1(dist, dens)
    centers = _stage2(scores)
    merged = _stage3(x, dist, centers.reshape(B, 1, KTOT))
    return centers, merged
